# baseline (speedup 1.0000x reference)
import functools

import jax
import jax.numpy as jnp
from jax import lax
from jax.experimental import pallas as pl
from jax.experimental.pallas import tpu as pltpu
from jax.experimental.pallas import tpu_sc as plsc

D = 1024
NC = 2
NS = 16
NW = NC * NS
CHUNK = 32
NBUF = 3


def kernel(position_ids, table):
    batch, seq = position_ids.shape
    total = batch * seq
    per_w = total // NW
    n_chunk = per_w // CHUNK
    main_iters = n_chunk // NBUF
    idx = position_ids.reshape(total)

    mesh = plsc.VectorSubcoreMesh(core_axis_name="c", subcore_axis_name="s")

    @functools.partial(
        pl.kernel,
        out_type=jax.ShapeDtypeStruct((total, D), jnp.float32),
        mesh=mesh,
        scratch_types=(
            [pltpu.VMEM((per_w,), jnp.int32)]
            + [pltpu.VMEM((CHUNK, D), jnp.float32) for _ in range(NBUF)]
            + [pltpu.SemaphoreType.DMA for _ in range(2 * NBUF)]
        ),
    )
    def gather_kernel(idx_hbm, table_hbm, out_hbm, idx_v, *rest):
        bufs = rest[:NBUF]
        gsem = rest[NBUF:2 * NBUF]
        wsem = rest[2 * NBUF:]
        wid = lax.axis_index("s") * NC + lax.axis_index("c")
        base = wid * per_w
        pltpu.sync_copy(idx_hbm.at[pl.ds(base, per_w)], idx_v)

        def gather(c, b):
            return pltpu.make_async_copy(
                table_hbm.at[idx_v.at[pl.ds(c * CHUNK, CHUNK)]],
                bufs[b], gsem[b])

        def write(c, b):
            return pltpu.make_async_copy(
                bufs[b], out_hbm.at[pl.ds(base + c * CHUNK, CHUNK)], wsem[b])

        def step(i, b):
            pb = (b - 1) % NBUF

            @pl.when((i >= 1) & (i - 1 + NBUF < n_chunk))
            def _():
                write(i - 1, pb).wait()
                gather(i - 1 + NBUF, pb).start()

            gather(i, b).wait()
            write(i, b).start()

        for b in range(NBUF):
            gather(b, b).start()

        @pl.loop(0, main_iters)
        def _(j):
            for b in range(NBUF):
                step(j * NBUF + b, b)

        for i in range(main_iters * NBUF, n_chunk):
            gather(i, i % NBUF).wait()
            write(i, i % NBUF).start()
        for i in range(n_chunk - NBUF, n_chunk):
            write(i, i % NBUF).wait()

    out = gather_kernel(idx, table)
    return out.reshape(batch, seq, D)
